# Initial kernel scaffold; baseline (speedup 1.0000x reference)
import jax
import jax.numpy as jnp
from jax.experimental import pallas as pl


def kernel(xyz):
    raise NotImplementedError("write your pallas kernel here")


# speedup vs baseline: 3.3777x; 3.3777x over previous
import jax
import jax.numpy as jnp
from jax import lax
from jax.experimental import pallas as pl

B, N, C = 8, 8192, 12
G, K = 128, 32


def _fps_body(coords_ref, cents_ref):
    cx = coords_ref[0]
    cy = coords_ref[1]
    cz = coords_ref[2]
    iota = lax.broadcasted_iota(jnp.int32, (B, N), 1)

    def step(s, carry):
        distance, farthest = carry
        cents_ref[pl.ds(s, 1)] = farthest[None]
        m = iota == farthest
        ctx = jnp.sum(jnp.where(m, cx, 0.0), axis=1, keepdims=True)
        cty = jnp.sum(jnp.where(m, cy, 0.0), axis=1, keepdims=True)
        ctz = jnp.sum(jnp.where(m, cz, 0.0), axis=1, keepdims=True)
        dist = (cx - ctx) ** 2 + (cy - cty) ** 2 + (cz - ctz) ** 2
        distance = jnp.minimum(distance, dist)
        maxv = jnp.max(distance, axis=1, keepdims=True)
        farthest = jnp.min(
            jnp.where(distance == maxv, iota, N), axis=1, keepdims=True
        ).astype(jnp.int32)
        return distance, farthest

    init = (jnp.full((B, N), 1e10, jnp.float32), jnp.zeros((B, 1), jnp.int32))
    lax.fori_loop(0, G, step, init)


def _group_body(xyzt_ref, cents_ref, nb_ref, center_ref):
    xr = xyzt_ref[0]
    cg = cents_ref[0]
    lane = lax.broadcasted_iota(jnp.int32, (G, N), 1)
    dn = (((1,), (1,)), ((), ()))
    oh = jnp.where(lane == cg, 1.0, 0.0)
    center = lax.dot_general(
        oh, xr, dn, precision=lax.Precision.HIGHEST,
        preferred_element_type=jnp.float32)
    center_ref[0] = center
    xn = jnp.sum(xr * xr, axis=0, keepdims=True)
    cn = jnp.sum(center * center, axis=1, keepdims=True)
    dots = lax.dot_general(
        center, xr, (((1,), (0,)), ((), ())),
        precision=lax.Precision.HIGHEST,
        preferred_element_type=jnp.float32)
    d2 = xn + cn - 2.0 * dots

    def step(k, d2):
        minv = jnp.min(d2, axis=1, keepdims=True)
        idxk = jnp.min(jnp.where(d2 == minv, lane, N), axis=1, keepdims=True)
        m = lane == idxk
        mf = jnp.where(m, 1.0, 0.0)
        nb = lax.dot_general(
            mf, xr, dn, precision=lax.Precision.HIGHEST,
            preferred_element_type=jnp.float32) - center
        nb_ref[0, pl.ds(k, 1)] = nb[None]
        return jnp.where(m, jnp.inf, d2)

    lax.fori_loop(0, K, step, d2)


def kernel(xyz):
    xyzt = jnp.transpose(xyz, (0, 2, 1))
    coords = jnp.transpose(xyz[:, :, 9:12], (2, 0, 1))

    cents = pl.pallas_call(
        _fps_body,
        out_shape=jax.ShapeDtypeStruct((G, B, 1), jnp.int32),
    )(coords)

    cents3 = jnp.transpose(cents, (1, 0, 2))

    nb, center = pl.pallas_call(
        _group_body,
        grid=(B,),
        in_specs=[
            pl.BlockSpec((1, C, N), lambda b: (b, 0, 0)),
            pl.BlockSpec((1, G, 1), lambda b: (b, 0, 0)),
        ],
        out_specs=[
            pl.BlockSpec((1, K, G, C), lambda b: (b, 0, 0, 0)),
            pl.BlockSpec((1, G, C), lambda b: (b, 0, 0)),
        ],
        out_shape=[
            jax.ShapeDtypeStruct((B, K, G, C), jnp.float32),
            jax.ShapeDtypeStruct((B, G, C), jnp.float32),
        ],
    )(xyzt, cents3)

    return jnp.transpose(nb, (0, 2, 1, 3)), center


# speedup vs baseline: 5.0479x; 1.4945x over previous
import functools

import jax
import jax.numpy as jnp
import numpy as np
from jax import lax
from jax.experimental import pallas as pl
from jax.experimental.pallas import tpu as pltpu
from jax.experimental.pallas import tpu_sc as plsc

B, N, C = 8, 8192, 12
G, K = 128, 32
CP = 16
NROW = B * G
NBLK = 32
BLK = N // NBLK
NW = 32
RPW = NROW // NW
NV = N // 16
CAPW = N
CAPC = 1024
FBIG = np.float32(3e38)
IBIG = np.int32(1 << 30)


def _fps_body(coords_ref, cents_ref):
    cx = coords_ref[0]
    cy = coords_ref[1]
    cz = coords_ref[2]
    iota = lax.broadcasted_iota(jnp.int32, (B, N), 1)

    def step(s, carry):
        distance, farthest = carry
        cents_ref[pl.ds(s, 1)] = farthest[None]
        m = iota == farthest
        ctx = jnp.sum(jnp.where(m, cx, 0.0), axis=1, keepdims=True)
        cty = jnp.sum(jnp.where(m, cy, 0.0), axis=1, keepdims=True)
        ctz = jnp.sum(jnp.where(m, cz, 0.0), axis=1, keepdims=True)
        dist = (cx - ctx) ** 2 + (cy - cty) ** 2 + (cz - ctz) ** 2
        distance = jnp.minimum(distance, dist)
        maxv = jnp.max(distance, axis=1, keepdims=True)
        farthest = jnp.min(
            jnp.where(distance == maxv, iota, N), axis=1, keepdims=True
        ).astype(jnp.int32)
        return distance, farthest

    init = (jnp.full((B, N), 1e10, jnp.float32), jnp.zeros((B, 1), jnp.int32))
    lax.fori_loop(0, G, step, init)


def _dist_body(xyzt_ref, cents_ref, center_ref, d2_ref, thr_ref):
    xr = xyzt_ref[0]
    cg = cents_ref[0]
    lane = lax.broadcasted_iota(jnp.int32, (G, N), 1)
    oh = jnp.where(lane == cg, 1.0, 0.0)
    center = lax.dot_general(
        oh, xr, (((1,), (1,)), ((), ())),
        precision=lax.Precision.HIGHEST,
        preferred_element_type=jnp.float32)
    center_ref[0] = center
    xn = jnp.sum(xr * xr, axis=0, keepdims=True)
    cn = jnp.sum(center * center, axis=1, keepdims=True)
    dots = lax.dot_general(
        center, xr, (((1,), (0,)), ((), ())),
        precision=lax.Precision.HIGHEST,
        preferred_element_type=jnp.float32)
    d2 = xn + cn - 2.0 * dots
    d2_ref[0] = d2
    t = jnp.full((G, 1), -jnp.inf, jnp.float32)
    for j in range(NBLK):
        bm = jnp.min(d2[:, j * BLK:(j + 1) * BLK], axis=1, keepdims=True)
        t = jnp.maximum(t, bm)
    thr_ref[0] = t


def _sc_body(d2_hbm, thr_hbm, xyzp_hbm, centp_hbm, out_hbm,
             dbuf0, dbuf1, wval, widx, cval, cidx, knn, gknn, nbuf, obuf,
             cent_vm, thr_vm, sem0, sem1, semg):
    wid = lax.axis_index("s") * 2 + lax.axis_index("c")
    base = wid * RPW
    iota16 = lax.iota(jnp.int32, 16)
    lane0 = iota16 == 0
    infv = jnp.full((16,), jnp.inf, jnp.float32)

    pltpu.sync_copy(thr_hbm.at[pl.ds(base, RPW)], thr_vm.at[pl.ds(0, RPW)])
    pltpu.sync_copy(centp_hbm.at[pl.ds(base * CP, RPW * CP)], cent_vm)

    def process(row, jloc, dbuf):
        tvec = thr_vm[pl.ds((jloc >> 4) << 4, 16)]
        ts = jnp.sum(jnp.where(iota16 == (jloc & 15), tvec, 0.0))
        thrv = jnp.full((16,), ts)

        def fbody(i, cnt):
            v = dbuf[pl.ds(i * 16, 16)]
            m = v <= thrv
            c = plsc.all_reduce_population_count(m)
            pos = cnt + iota16
            plsc.store_scatter(wval, [pos], jnp.where(m, v, FBIG))
            plsc.store_scatter(widx, [pos], jnp.where(m, iota16 + i * 16, IBIG))
            return cnt + jnp.minimum(c, 1) * 16

        cnt = lax.fori_loop(0, NV, fbody, jnp.zeros((16,), jnp.int32))
        m1 = jnp.max(cnt) >> 4

        def pre(j, _):
            cval[pl.ds(j * 16, 16)] = infv
            return 0
        lax.fori_loop(0, CAPC // 16, pre, 0)

        def cbody(j, cnt2):
            wv = wval[pl.ds(j * 16, 16)]
            wm = wv < FBIG
            wi = widx[pl.ds(j * 16, 16)]
            wmi = wm.astype(jnp.int32)
            exc = plsc.cumsum(wmi) - wmi
            pos2 = jnp.minimum(cnt2 + exc, CAPC - 1)
            plsc.store_scatter(cval, [pos2], wv, mask=wm)
            plsc.store_scatter(cidx, [pos2], wi, mask=wm)
            return cnt2 + plsc.all_reduce_population_count(wm)

        lax.fori_loop(0, m1, cbody, jnp.zeros((16,), jnp.int32))
        m2 = CAPC // 16

        def ebody(k, _):
            def scan1(j, acc):
                return jnp.minimum(acc, cval[pl.ds(j * 16, 16)])
            mv = lax.fori_loop(0, m2, scan1, infv)
            s = jnp.min(mv)
            msplat = jnp.full((16,), s)

            def scan2(j, acc):
                v = cval[pl.ds(j * 16, 16)]
                p = jnp.where(v == msplat, iota16 + j * 16, IBIG)
                return jnp.minimum(acc, p)
            pv = lax.fori_loop(0, m2, scan2, jnp.full((16,), IBIG))
            p = jnp.min(pv)
            psplat = jnp.full((16,), p)
            ivec = cidx[pl.ds((p >> 4) << 4, 16)]
            iv = jnp.sum(jnp.where(iota16 == (p & 15), ivec, 0))
            plsc.store_scatter(knn, [jnp.full((16,), k)],
                               jnp.full((16,), iv), mask=lane0)
            plsc.store_scatter(cval, [psplat], infv, mask=lane0)
            return 0

        lax.fori_loop(0, K, ebody, 0)

        boff = (row >> 7) << 13
        gknn[pl.ds(0, 16)] = (knn[pl.ds(0, 16)] + boff) >> 3
        gknn[pl.ds(16, 16)] = (knn[pl.ds(16, 16)] + boff) >> 3
        pltpu.async_copy(xyzp_hbm.at[gknn], nbuf, semg).wait()
        cvec = cent_vm[pl.ds(jloc * 16, 16)]
        for nn in range(K):
            kvec = knn[pl.ds((nn // 16) * 16, 16)]
            sel = jnp.sum(jnp.where(iota16 == nn % 16, kvec, 0))
            ivec = jnp.full((16,), (sel & 7) * 16) + iota16
            v = plsc.load_gather(nbuf, [jnp.full((16,), nn, jnp.int32), ivec])
            obuf[nn] = v - cvec
        pltpu.sync_copy(obuf, out_hbm.at[row])

    pltpu.async_copy(d2_hbm.at[base], dbuf0, sem0)

    def dbody(i, _):
        r0 = base + 2 * i
        r1 = r0 + 1
        r2 = jnp.minimum(r0 + 2, base + RPW - 1)
        pltpu.async_copy(d2_hbm.at[r1], dbuf1, sem1)
        pltpu.make_async_copy(d2_hbm.at[r0], dbuf0, sem0).wait()
        process(r0, 2 * i, dbuf0)
        pltpu.async_copy(d2_hbm.at[r2], dbuf0, sem0)
        pltpu.make_async_copy(d2_hbm.at[r1], dbuf1, sem1).wait()
        process(r1, 2 * i + 1, dbuf1)
        return 0

    lax.fori_loop(0, RPW // 2, dbody, 0)
    pltpu.make_async_copy(d2_hbm.at[base], dbuf0, sem0).wait()


def kernel(xyz):
    xyzt = jnp.transpose(xyz, (0, 2, 1))
    coords = jnp.transpose(xyz[:, :, 9:12], (2, 0, 1))

    cents = pl.pallas_call(
        _fps_body,
        out_shape=jax.ShapeDtypeStruct((G, B, 1), jnp.int32),
    )(coords)
    cents3 = jnp.transpose(cents, (1, 0, 2))

    center, d2, thr = pl.pallas_call(
        _dist_body,
        grid=(B,),
        in_specs=[
            pl.BlockSpec((1, C, N), lambda b: (b, 0, 0)),
            pl.BlockSpec((1, G, 1), lambda b: (b, 0, 0)),
        ],
        out_specs=[
            pl.BlockSpec((1, G, C), lambda b: (b, 0, 0)),
            pl.BlockSpec((1, G, N), lambda b: (b, 0, 0)),
            pl.BlockSpec((1, G, 1), lambda b: (b, 0, 0)),
        ],
        out_shape=[
            jax.ShapeDtypeStruct((B, G, C), jnp.float32),
            jax.ShapeDtypeStruct((B, G, N), jnp.float32),
            jax.ShapeDtypeStruct((B, G, 1), jnp.float32),
        ],
    )(xyzt, cents3)

    d2r = d2.reshape(NROW, N)
    thr1 = thr.reshape(NROW)
    xyzp = jnp.pad(xyz.reshape(B * N, C), ((0, 0), (0, CP - C)))
    xyzp = xyzp.reshape(B * N * CP // 128, 128)
    centp = jnp.pad(center.reshape(NROW, C), ((0, 0), (0, CP - C))).reshape(-1)

    mesh = plsc.VectorSubcoreMesh(core_axis_name="c", subcore_axis_name="s")
    nb = pl.kernel(
        _sc_body,
        out_type=jax.ShapeDtypeStruct((NROW, K, CP), jnp.float32),
        mesh=mesh,
        compiler_params=pltpu.CompilerParams(needs_layout_passes=False),
        scratch_types=[
            pltpu.VMEM((N,), jnp.float32),
            pltpu.VMEM((N,), jnp.float32),
            pltpu.VMEM((CAPW,), jnp.float32),
            pltpu.VMEM((CAPW,), jnp.int32),
            pltpu.VMEM((CAPC,), jnp.float32),
            pltpu.VMEM((CAPC,), jnp.int32),
            pltpu.VMEM((128,), jnp.int32),
            pltpu.VMEM((K,), jnp.int32),
            pltpu.VMEM((K, 128), jnp.float32),
            pltpu.VMEM((K, CP), jnp.float32),
            pltpu.VMEM((RPW * CP,), jnp.float32),
            pltpu.VMEM((128,), jnp.float32),
            pltpu.SemaphoreType.DMA,
            pltpu.SemaphoreType.DMA,
            pltpu.SemaphoreType.DMA,
        ],
    )(d2r, thr1, xyzp, centp)

    neighborhood = nb[:, :, :C].reshape(B, G, K, C)
    return neighborhood, center
